# Initial kernel scaffold; baseline (speedup 1.0000x reference)
import jax
import jax.numpy as jnp
from jax import lax
from jax.experimental import pallas as pl
from jax.experimental.pallas import tpu as pltpu
from jax.experimental.pallas import tpu_sc as plsc

N = 10000
E = 320000
D = 128
ALPHA = 0.2

NC = 2
NS = 16
L = 16
NW = NC * NS
EPW = E // NW
C = 80
NCHUNK = EPW // C
RPW = N // NS
ZR = 125


def _dense_body(h_ref, w_ref, a2_ref, wh_ref, wh12_ref):
    wh = jnp.dot(h_ref[...], w_ref[...],
                 preferred_element_type=jnp.float32,
                 precision=lax.Precision.HIGHEST)
    wh_ref[...] = wh
    wh12_ref[...] = lax.dot_general(
        a2_ref[...], wh, (((0,), (1,)), ((), ())),
        preferred_element_type=jnp.float32,
        precision=lax.Precision.HIGHEST)


def _dense(h, W, a2):
    return pl.pallas_call(
        _dense_body,
        out_shape=[
            jax.ShapeDtypeStruct((N, D), jnp.float32),
            jax.ShapeDtypeStruct((2, N), jnp.float32),
        ],
    )(h, W, a2)


def _edge_body(wh_hbm, wh12_hbm, edges_hbm, unnorm_hbm, rowsum_hbm,
               wh1_v, wh2_v, rs_v, row_v, col_v, v_v, rows_v, zbuf_v,
               unnorm_sh, sem):
    cid = lax.axis_index("c")
    sid = lax.axis_index("s")
    wid = sid * NC + cid

    zeros = jnp.zeros((L,), jnp.float32)

    @pl.loop(0, N // L)
    def _(i):
        rs_v[pl.ds(i * L, L)] = zeros

    @pl.loop(0, ZR)
    def _(i):
        for j in range(D // L):
            zbuf_v[i, pl.ds(j * L, L)] = zeros

    for k in range(RPW // ZR):
        pltpu.sync_copy(zbuf_v, unnorm_sh.at[pl.ds(sid * RPW + k * ZR, ZR)])

    pltpu.sync_copy(wh12_hbm.at[0], wh1_v)
    pltpu.sync_copy(wh12_hbm.at[1], wh2_v)

    plsc.subcore_barrier()

    base_e = wid * EPW

    @pl.loop(0, NCHUNK)
    def _(cidx):
        eb = base_e + cidx * C
        pltpu.sync_copy(edges_hbm.at[0, pl.ds(eb, C)], row_v)
        pltpu.sync_copy(edges_hbm.at[1, pl.ds(eb, C)], col_v)
        gather = pltpu.async_copy(wh_hbm.at[col_v], rows_v, sem)

        for g in range(C // L):
            r16 = row_v[pl.ds(g * L, L)]
            c16 = col_v[pl.ds(g * L, L)]
            ee = plsc.load_gather(wh1_v, [r16]) + plsc.load_gather(wh2_v, [c16])
            ee = jnp.where(ee >= 0, ee, ALPHA * ee)
            vv = jnp.exp(ee)
            v_v[pl.ds(g * L, L)] = vv
            plsc.addupdate_scatter(rs_v, [r16], vv)

        gather.wait()

        for i in range(C):
            b = plsc.load_gather(v_v, [jnp.full((L,), i, jnp.int32)])
            for j in range(D // L):
                sl = (i, pl.ds(j * L, L))
                rows_v[sl] = rows_v[sl] * b

        pltpu.sync_copy(rows_v, unnorm_sh.at[row_v], add=True)

    plsc.subcore_barrier()

    pltpu.sync_copy(unnorm_sh.at[pl.ds(sid * RPW, RPW)],
                    unnorm_hbm.at[cid, pl.ds(sid * RPW, RPW)])
    pltpu.sync_copy(rs_v, rowsum_hbm.at[wid])


def _edge_pass(wh, wh12, edges):
    mesh = plsc.VectorSubcoreMesh(core_axis_name="c", subcore_axis_name="s")
    kern = pl.kernel(
        _edge_body,
        out_type=[
            jax.ShapeDtypeStruct((NC, N, D), jnp.float32),
            jax.ShapeDtypeStruct((NW, N), jnp.float32),
        ],
        mesh=mesh,
        scratch_types=[
            pltpu.VMEM((N,), jnp.float32),
            pltpu.VMEM((N,), jnp.float32),
            pltpu.VMEM((N,), jnp.float32),
            pltpu.VMEM((C,), jnp.int32),
            pltpu.VMEM((C,), jnp.int32),
            pltpu.VMEM((C,), jnp.float32),
            pltpu.VMEM((C, D), jnp.float32),
            pltpu.VMEM((ZR, D), jnp.float32),
            pltpu.VMEM_SHARED((N, D), jnp.float32),
            pltpu.SemaphoreType.DMA,
        ],
    )
    return kern(wh, wh12, edges)


def _combine_body(u_ref, rs_ref, out_ref):
    rs = jnp.sum(rs_ref[...], axis=0)
    u = u_ref[0] + u_ref[1]
    rs_col = rs[:, None]
    safe = jnp.where(rs_col > 0, rs_col, 1.0)
    x = jnp.where(rs_col > 0, u / safe, 0.0)
    out_ref[...] = jnp.where(x > 0, x, jnp.expm1(x))


def _combine(unnorm, rowsum):
    return pl.pallas_call(
        _combine_body,
        out_shape=jax.ShapeDtypeStruct((N, D), jnp.float32),
    )(unnorm, rowsum)


def kernel(h, edge_index, W, a):
    a2 = jnp.concatenate([a[:D], a[D:]], axis=1)
    wh, wh12 = _dense(h, W, a2)
    unnorm, rowsum = _edge_pass(wh, wh12, edge_index)
    return _combine(unnorm, rowsum)


# speedup vs baseline: 23.3159x; 23.3159x over previous
import jax
import jax.numpy as jnp
from jax import lax
from jax.experimental import pallas as pl
from jax.experimental.pallas import tpu as pltpu
from jax.experimental.pallas import tpu_sc as plsc

N = 10000
E = 320000
D = 128
ALPHA = 0.2

NC = 2
NS = 16
L = 16
NW = NC * NS
EPW = E // NW
C = 80
NCHUNK = EPW // C
N_PAD = 10240
RPW = N_PAD // NS


def _dense_body(h_ref, w_ref, a2_ref, wh_ref, wh12_ref):
    wh = jnp.dot(h_ref[...], w_ref[...],
                 preferred_element_type=jnp.float32,
                 precision=lax.Precision.HIGHEST)
    wh_ref[...] = wh
    wh12_ref[...] = lax.dot_general(
        a2_ref[...], wh, (((0,), (1,)), ((), ())),
        preferred_element_type=jnp.float32,
        precision=lax.Precision.HIGHEST)


def _dense(h, W, a2):
    return pl.pallas_call(
        _dense_body,
        out_shape=[
            jax.ShapeDtypeStruct((N, D), jnp.float32),
            jax.ShapeDtypeStruct((2, N), jnp.float32),
        ],
    )(h, W, a2)


def _edge_body(wh_hbm, wh12_hbm, erow_hbm, ecol_hbm, unnorm_hbm, rowsum_hbm,
               wh12_v, rs_v, row_v, col_v, v_v, rows_v,
               unnorm_sh, sem):
    cid = lax.axis_index("c")
    sid = lax.axis_index("s")
    wid = sid * NC + cid

    zeros = jnp.zeros((L,), jnp.float32)
    zero16i = jnp.zeros((L,), jnp.int32)
    one16i = jnp.ones((L,), jnp.int32)

    @pl.loop(0, N_PAD // L)
    def _(i):
        rs_v[pl.ds(i * L, L)] = zeros

    @pl.loop(0, C)
    def _(i):
        for j in range(D // L):
            rows_v[i, pl.ds(j * L, L)] = zeros

    for k in range(RPW // C):
        pltpu.sync_copy(rows_v, unnorm_sh.at[pl.ds(sid * RPW + k * C, C)])

    pltpu.sync_copy(wh12_hbm, wh12_v)

    plsc.subcore_barrier()

    base_e = wid * EPW

    @pl.loop(0, NCHUNK)
    def _(cidx):
        eb = base_e + cidx * C
        pltpu.sync_copy(erow_hbm.at[pl.ds(eb, C)], row_v)
        pltpu.sync_copy(ecol_hbm.at[pl.ds(eb, C)], col_v)
        gather = pltpu.async_copy(wh_hbm.at[col_v], rows_v, sem)

        for g in range(C // L):
            r16 = row_v[pl.ds(g * L, L)]
            c16 = col_v[pl.ds(g * L, L)]
            ee = (plsc.load_gather(wh12_v, [zero16i, r16])
                  + plsc.load_gather(wh12_v, [one16i, c16]))
            ee = jnp.where(ee >= 0, ee, ALPHA * ee)
            vv = jnp.exp(ee)
            v_v[pl.ds(g * L, L)] = vv
            if g == 0:
                v_v[pl.ds(C, L)] = vv
            plsc.addupdate_scatter(rs_v, [r16], vv)

        gather.wait()

        for i in range(C):
            b = plsc.load_gather(v_v, [jnp.full((L,), i if i else C, jnp.int32)])
            for j in range(D // L):
                sl = (i, pl.ds(j * L, L))
                rows_v[sl] = rows_v[sl] * b

        pltpu.sync_copy(rows_v, unnorm_sh.at[row_v], add=True)

    plsc.subcore_barrier()

    pltpu.sync_copy(unnorm_sh.at[pl.ds(sid * RPW, RPW)],
                    unnorm_hbm.at[cid, pl.ds(sid * RPW, RPW)])
    pltpu.sync_copy(rs_v, rowsum_hbm.at[pl.ds(wid * N_PAD, N_PAD)])


def _edge_pass(wh, wh12, erow, ecol):
    mesh = plsc.VectorSubcoreMesh(core_axis_name="c", subcore_axis_name="s")
    kern = pl.kernel(
        _edge_body,
        out_type=[
            jax.ShapeDtypeStruct((NC, N_PAD, D), jnp.float32),
            jax.ShapeDtypeStruct((NW * N_PAD,), jnp.float32),
        ],
        mesh=mesh,
        compiler_params=pltpu.CompilerParams(needs_layout_passes=False),
        scratch_types=[
            pltpu.VMEM((2, N), jnp.float32),
            pltpu.VMEM((N_PAD,), jnp.float32),
            pltpu.VMEM((C,), jnp.int32),
            pltpu.VMEM((C,), jnp.int32),
            pltpu.VMEM((C + L,), jnp.float32),
            pltpu.VMEM((C, D), jnp.float32),
            pltpu.VMEM_SHARED((N_PAD, D), jnp.float32),
            pltpu.SemaphoreType.DMA,
        ],
    )
    return kern(wh, wh12, erow, ecol)


def _combine_body(u_ref, rs_ref, out_ref):
    rs = jnp.sum(rs_ref[...], axis=0)
    u = u_ref[0] + u_ref[1]
    rs_col = rs[:, None]
    safe = jnp.where(rs_col > 0, rs_col, 1.0)
    x = jnp.where(rs_col > 0, u / safe, 0.0)
    out_ref[...] = jnp.where(x > 0, x, jnp.exp(jnp.minimum(x, 0.0)) - 1.0)


def _combine(unnorm, rowsum2d):
    return pl.pallas_call(
        _combine_body,
        out_shape=jax.ShapeDtypeStruct((N_PAD, D), jnp.float32),
    )(unnorm, rowsum2d)


def kernel(h, edge_index, W, a):
    a2 = jnp.concatenate([a[:D], a[D:]], axis=1)
    wh, wh12 = _dense(h, W, a2)
    unnorm, rowsum = _edge_pass(wh, wh12, edge_index[0], edge_index[1])
    out = _combine(unnorm, rowsum.reshape(NW, N_PAD))
    return out[:N]
